# baseline (speedup 1.0000x reference)
import functools

import jax
import jax.numpy as jnp
from jax import lax
from jax.experimental import pallas as pl
from jax.experimental.pallas import tpu as pltpu
from jax.experimental.pallas import tpu_sc as plsc

NC = 2
NS = 16
NW = NC * NS
CHUNK = 128
EPS = 1e-5

F32 = jnp.float32


def _sc_mesh():
    return plsc.VectorSubcoreMesh(
        core_axis_name="c", subcore_axis_name="s",
        num_cores=NC, num_subcores=NS)


NPD = NS * 640


def _deg_body(cpt, dsti, ones_hbm, zer_hbm, out, dst_v, ones_v, dacc):
    c = lax.axis_index("c")
    s = lax.axis_index("s")
    wid = s * NC + c
    pltpu.sync_copy(zer_hbm, dacc.at[pl.ds(s * 640, 640)])
    pltpu.sync_copy(ones_hbm, ones_v)
    pltpu.sync_copy(dsti.at[wid], dst_v)
    plsc.subcore_barrier()

    def body(ci, carry):
        pltpu.sync_copy(ones_v, dacc.at[dst_v.at[ci]], add=True)
        return carry

    lax.fori_loop(0, cpt, body, 0)
    plsc.subcore_barrier()
    pltpu.sync_copy(dacc.at[pl.ds(s * 640, 640)],
                    out.at[c, 0, pl.ds(s * 640, 640)])


def _sc_degree(cpt, dsti, ones_in, zer_in):
    f = pl.kernel(
        functools.partial(_deg_body, cpt),
        out_type=jax.ShapeDtypeStruct((NC, 1, NPD), F32),
        mesh=_sc_mesh(),
        scratch_types=[
            pltpu.VMEM((cpt, CHUNK), jnp.int32),
            pltpu.VMEM((CHUNK,), F32),
            pltpu.VMEM_SHARED((NPD,), F32),
        ],
    )
    return f(dsti, ones_in, zer_in)


def _agg_body(cpt, rpw, hs, srci, dsti, zer_hbm, out,
              src_v, dring, rows0, rows1, acc, semd0, semd1, semg0, semg1):
    c = lax.axis_index("c")
    s = lax.axis_index("s")
    wid = s * NC + c
    pltpu.sync_copy(zer_hbm, acc.at[pl.ds(s * rpw, rpw)])
    pltpu.sync_copy(srci.at[wid], src_v)
    plsc.subcore_barrier()

    pltpu.async_copy(dsti.at[wid, 0], dring.at[0], semd0)
    pltpu.async_copy(hs.at[src_v.at[0]], rows0, semg0)
    pltpu.async_copy(dsti.at[wid, 1], dring.at[1], semd1)
    pltpu.async_copy(hs.at[src_v.at[1]], rows1, semg1)

    def body(i, carry):
        c0 = 2 * i

        pltpu.make_async_copy(dsti.at[wid, c0], dring.at[0], semd0).wait()
        pltpu.make_async_copy(hs.at[src_v.at[c0]], rows0, semg0).wait()
        pltpu.sync_copy(rows0, acc.at[dring.at[0]], add=True)

        @pl.when(c0 + 2 < cpt)
        def _():
            pltpu.async_copy(dsti.at[wid, c0 + 2], dring.at[0], semd0)
            pltpu.async_copy(hs.at[src_v.at[c0 + 2]], rows0, semg0)

        pltpu.make_async_copy(dsti.at[wid, c0 + 1], dring.at[1], semd1).wait()
        pltpu.make_async_copy(hs.at[src_v.at[c0 + 1]], rows1, semg1).wait()
        pltpu.sync_copy(rows1, acc.at[dring.at[1]], add=True)

        @pl.when(c0 + 3 < cpt)
        def _():
            pltpu.async_copy(dsti.at[wid, c0 + 3], dring.at[1], semd1)
            pltpu.async_copy(hs.at[src_v.at[c0 + 3]], rows1, semg1)

        return carry

    lax.fori_loop(0, cpt // 2, body, 0)
    plsc.subcore_barrier()
    pltpu.sync_copy(acc.at[pl.ds(s * rpw, rpw)],
                    out.at[c, pl.ds(s * rpw, rpw)])


def _sc_aggregate(np_, cpt, hs, srci, dsti, zer_in):
    rpw = np_ // NS
    f = pl.kernel(
        functools.partial(_agg_body, cpt, rpw),
        out_type=jax.ShapeDtypeStruct((NC, np_, 128), F32),
        mesh=_sc_mesh(),
        scratch_types=[
            pltpu.VMEM((cpt, CHUNK), jnp.int32),
            pltpu.VMEM((8, CHUNK), jnp.int32),
            pltpu.VMEM((CHUNK, 128), F32),
            pltpu.VMEM((CHUNK, 128), F32),
            pltpu.VMEM_SHARED((np_, 128), F32),
            pltpu.SemaphoreType.DMA,
            pltpu.SemaphoreType.DMA,
            pltpu.SemaphoreType.DMA,
            pltpu.SemaphoreType.DMA,
        ],
    )
    return f(hs, srci, dsti, zer_in)


def _gelu(v):
    return 0.5 * v * (1.0 + lax.erf(v * (2.0 ** -0.5)))


def _dot(a, b):
    return jnp.dot(a, b, preferred_element_type=F32,
                   precision=lax.Precision.HIGHEST)


def _scale_body(x_ref, w_ref, d0_ref, d1_ref, hs_ref, dinv_ref):
    deg = d0_ref[0, 0, :] + d1_ref[0, 0, :] + 1.0
    dv = lax.rsqrt(deg)
    dinv_ref[0, 0, :] = dv
    hs_ref[...] = dv[:, None] * _dot(x_ref[...], w_ref[...])


def _tc_scale(np_, nblk, r, xp, w0big, deg0, deg1):
    f = pl.pallas_call(
        _scale_body,
        grid=(nblk,),
        in_specs=[
            pl.BlockSpec((r, 512), lambda i: (i, 0)),
            pl.BlockSpec((512, 128), lambda i: (0, 0)),
            pl.BlockSpec((1, 1, r), lambda i: (i, 0, 0)),
            pl.BlockSpec((1, 1, r), lambda i: (i, 0, 0)),
        ],
        out_specs=[
            pl.BlockSpec((r, 128), lambda i: (i, 0)),
            pl.BlockSpec((1, 1, r), lambda i: (i, 0, 0)),
        ],
        out_shape=[
            jax.ShapeDtypeStruct((np_, 128), F32),
            jax.ShapeDtypeStruct((nblk, 1, r), F32),
        ],
        compiler_params=pltpu.CompilerParams(
            dimension_semantics=("arbitrary",)),
    )
    return f(xp, w0big, deg0, deg1)


def _stats_body(g, r, parts_ref, hs_ref, dinv_ref, bt_ref, batch_ref,
                s_ref, stats_ref):
    i = pl.program_id(0)
    dv = dinv_ref[0, 0, :]
    sv = dv[:, None] * (parts_ref[0] + parts_ref[1] + hs_ref[...]) \
        + bt_ref[0, :][None, :]
    s_ref[...] = sv
    b = batch_ref[0, 0, :]
    oh = (lax.broadcasted_iota(jnp.int32, (g, r), 0) == b[None, :]) \
        .astype(F32)
    a1 = _dot(oh, sv)
    a2 = _dot(oh, sv * sv)
    cnt = jnp.sum(oh, axis=1)
    new = jnp.stack([a1, a2, jnp.broadcast_to(cnt[:, None], (g, 128))])

    @pl.when(i == 0)
    def _():
        stats_ref[...] = new

    @pl.when(i > 0)
    def _():
        stats_ref[...] += new


def _tc_stats(g, np_, nblk, r, parts, hs, dinv, bt, batch3):
    f = pl.pallas_call(
        functools.partial(_stats_body, g, r),
        grid=(nblk,),
        in_specs=[
            pl.BlockSpec((2, r, 128), lambda i: (0, i, 0)),
            pl.BlockSpec((r, 128), lambda i: (i, 0)),
            pl.BlockSpec((1, 1, r), lambda i: (i, 0, 0)),
            pl.BlockSpec((1, 128), lambda i: (0, 0)),
            pl.BlockSpec((1, 1, r), lambda i: (i, 0, 0)),
        ],
        out_specs=[
            pl.BlockSpec((r, 128), lambda i: (i, 0)),
            pl.BlockSpec((3, g, 128), lambda i: (0, 0, 0)),
        ],
        out_shape=[
            jax.ShapeDtypeStruct((np_, 128), F32),
            jax.ShapeDtypeStruct((3, g, 128), F32),
        ],
        compiler_params=pltpu.CompilerParams(
            dimension_semantics=("arbitrary",)),
    )
    return f(parts, hs, dinv, bt, batch3)


def _norm_scale_off(stats, gw, gb, gms):
    cnt = jnp.maximum(stats[2], 1.0)
    mean = stats[0] / cnt
    e2 = stats[1] / cnt
    ms = gms[0, :][None, :]
    var = e2 - mean * mean * ms * (2.0 - ms)
    rs = lax.rsqrt(var + EPS)
    w = gw[0, :][None, :]
    scale = w * rs
    off = gb[0, :][None, :] - w * ms * mean * rs
    return scale, off


def _row_gather(batch_ref, g, r, scale, off):
    b = batch_ref[0, 0, :]
    oht = (b[:, None] == lax.broadcasted_iota(jnp.int32, (r, g), 1)) \
        .astype(F32)
    return _dot(oht, scale), _dot(oht, off)


def _norm0_body(g, r, s_ref, stats_ref, batch_ref, dinv_ref,
                gw_ref, gb_ref, gms_ref, w1_ref, res_ref, hs1_ref):
    scale, off = _norm_scale_off(stats_ref[...], gw_ref, gb_ref, gms_ref)
    sc_r, off_r = _row_gather(batch_ref, g, r, scale, off)
    gv = _gelu(s_ref[...] * sc_r + off_r)
    res_ref[...] = gv
    hs1_ref[...] = dinv_ref[0, 0, :][:, None] * _dot(gv, w1_ref[...])


def _tc_norm0(g, np_, nblk, r, s, stats, batch3, dinv, gw, gb, gms, w1big):
    f = pl.pallas_call(
        functools.partial(_norm0_body, g, r),
        grid=(nblk,),
        in_specs=[
            pl.BlockSpec((r, 128), lambda i: (i, 0)),
            pl.BlockSpec((3, g, 128), lambda i: (0, 0, 0)),
            pl.BlockSpec((1, 1, r), lambda i: (i, 0, 0)),
            pl.BlockSpec((1, 1, r), lambda i: (i, 0, 0)),
            pl.BlockSpec((1, 128), lambda i: (0, 0)),
            pl.BlockSpec((1, 128), lambda i: (0, 0)),
            pl.BlockSpec((1, 128), lambda i: (0, 0)),
            pl.BlockSpec((128, 128), lambda i: (0, 0)),
        ],
        out_specs=[
            pl.BlockSpec((r, 128), lambda i: (i, 0)),
            pl.BlockSpec((r, 128), lambda i: (i, 0)),
        ],
        out_shape=[
            jax.ShapeDtypeStruct((np_, 128), F32),
            jax.ShapeDtypeStruct((np_, 128), F32),
        ],
        compiler_params=pltpu.CompilerParams(
            dimension_semantics=("arbitrary",)),
    )
    return f(s, stats, batch3, dinv, gw, gb, gms, w1big)


def _final_body(g, r, s_ref, stats_ref, batch_ref, res_ref,
                gw_ref, gb_ref, gms_ref, hw0_ref, hb0_ref, hw1_ref, hb1_ref,
                z_ref):
    scale, off = _norm_scale_off(stats_ref[...], gw_ref, gb_ref, gms_ref)
    sc_r, off_r = _row_gather(batch_ref, g, r, scale, off)
    h1 = _gelu(s_ref[...] * sc_r + off_r) + res_ref[...]
    t1 = _gelu(_dot(h1, hw0_ref[...]) + hb0_ref[0, :][None, :])
    z_ref[...] = _dot(t1, hw1_ref[...]) + hb1_ref[0, :][None, :]


def _tc_final(g, np_, nblk, r, s, stats, batch3, res0,
              gw, gb, gms, hw0big, hb0t, hw1big, hb1t):
    f = pl.pallas_call(
        functools.partial(_final_body, g, r),
        grid=(nblk,),
        in_specs=[
            pl.BlockSpec((r, 128), lambda i: (i, 0)),
            pl.BlockSpec((3, g, 128), lambda i: (0, 0, 0)),
            pl.BlockSpec((1, 1, r), lambda i: (i, 0, 0)),
            pl.BlockSpec((r, 128), lambda i: (i, 0)),
            pl.BlockSpec((1, 128), lambda i: (0, 0)),
            pl.BlockSpec((1, 128), lambda i: (0, 0)),
            pl.BlockSpec((1, 128), lambda i: (0, 0)),
            pl.BlockSpec((128, 128), lambda i: (0, 0)),
            pl.BlockSpec((1, 128), lambda i: (0, 0)),
            pl.BlockSpec((128, 128), lambda i: (0, 0)),
            pl.BlockSpec((1, 128), lambda i: (0, 0)),
        ],
        out_specs=[pl.BlockSpec((r, 128), lambda i: (i, 0))],
        out_shape=[jax.ShapeDtypeStruct((np_, 128), F32)],
        compiler_params=pltpu.CompilerParams(
            dimension_semantics=("arbitrary",)),
    )
    return f(s, stats, batch3, res0, gw, gb, gms, hw0big, hb0t, hw1big, hb1t)[0]


def kernel(x, batch, edge_index, W0, b0, gn0_w, gn0_b, gn0_ms,
           W1, b1, gn1_w, gn1_b, gn1_ms, hW0, hb0, hW1, hb1):
    n, t, d = x.shape
    h = W0.shape[1]
    e = edge_index.shape[1]
    g = 16
    out_f = hW1.shape[1]

    r = 128
    np_ = ((n + 1 + r - 1) // r) * r
    nblk = np_ // r
    cpt = (e + NW * CHUNK - 1) // (NW * CHUNK)
    cpt = cpt + (cpt % 2)
    ept = cpt * CHUNK
    e_pad = ept * NW

    x2 = x.reshape(n, t * d)
    xp = jnp.zeros((np_, t * d), F32).at[:n].set(x2)
    batchp = jnp.full((np_,), g, jnp.int32).at[:n].set(batch.astype(jnp.int32))
    batch3 = batchp.reshape(nblk, 1, r)
    ei = edge_index.astype(jnp.int32)
    srcp = jnp.full((e_pad,), n, jnp.int32).at[:e].set(ei[0])
    dstp = jnp.full((e_pad,), n, jnp.int32).at[:e].set(ei[1])
    srci = srcp.reshape(NW, cpt, CHUNK)
    dsti = dstp.reshape(NW, cpt, CHUNK)

    eye_t = jnp.eye(t, dtype=F32)
    w0big = jnp.kron(eye_t, W0)
    w1big = jnp.kron(eye_t, W1)
    hw0big = jnp.kron(eye_t, hW0)
    hw1big = jnp.zeros((t * h, 128), F32).at[:, :t * out_f].set(
        jnp.kron(eye_t, hW1))
    b0t = jnp.tile(b0, t).reshape(1, t * h)
    b1t = jnp.tile(b1, t).reshape(1, t * h)
    gw0 = jnp.tile(gn0_w, t).reshape(1, t * h)
    gb0 = jnp.tile(gn0_b, t).reshape(1, t * h)
    gm0 = jnp.tile(gn0_ms, t).reshape(1, t * h)
    gw1 = jnp.tile(gn1_w, t).reshape(1, t * h)
    gb1 = jnp.tile(gn1_b, t).reshape(1, t * h)
    gm1 = jnp.tile(gn1_ms, t).reshape(1, t * h)
    hb0t = jnp.tile(hb0, t).reshape(1, t * h)
    hb1t = jnp.zeros((1, 128), F32).at[0, :t * out_f].set(jnp.tile(hb1, t))

    rpw = np_ // NS
    zer128 = jnp.zeros((rpw, 128), F32)
    zer1 = jnp.zeros((640,), F32)
    ones1 = jnp.ones((CHUNK,), F32)

    degp = _sc_degree(cpt, dsti, ones1, zer1)
    deg0 = degp[0, 0, :np_].reshape(nblk, 1, r)
    deg1 = degp[1, 0, :np_].reshape(nblk, 1, r)

    hs0, dinv = _tc_scale(np_, nblk, r, xp, w0big, deg0, deg1)
    parts0 = _sc_aggregate(np_, cpt, hs0, srci, dsti, zer128)
    s0, stats0 = _tc_stats(g, np_, nblk, r, parts0, hs0, dinv, b0t, batch3)
    res0, hs1 = _tc_norm0(g, np_, nblk, r, s0, stats0, batch3, dinv,
                          gw0, gb0, gm0, w1big)
    parts1 = _sc_aggregate(np_, cpt, hs1, srci, dsti, zer128)
    s1, stats1 = _tc_stats(g, np_, nblk, r, parts1, hs1, dinv, b1t, batch3)
    z = _tc_final(g, np_, nblk, r, s1, stats1, batch3, res0,
                  gw1, gb1, gm1, hw0big, hb0t, hw1big, hb1t)

    return z[:n, :t * out_f].reshape(n, t, out_f)


# speedup vs baseline: 1.9792x; 1.9792x over previous
import functools

import jax
import jax.numpy as jnp
from jax import lax
from jax.experimental import pallas as pl
from jax.experimental.pallas import tpu as pltpu
from jax.experimental.pallas import tpu_sc as plsc

NC = 2
NS = 16
NW = NC * NS
CHUNK = 128
EPS = 1e-5

F32 = jnp.float32


def _sc_mesh():
    return plsc.VectorSubcoreMesh(
        core_axis_name="c", subcore_axis_name="s",
        num_cores=NC, num_subcores=NS)


NPD = NS * 640


def _deg_body(cpt0, cpt1, dsti, ones_hbm, zer_hbm, out, dst_v, ones_v, dacc):
    c = lax.axis_index("c")
    s = lax.axis_index("s")
    wid = s * NC + c
    nch = jnp.where(c == 0, cpt0, cpt1)
    pltpu.sync_copy(zer_hbm, dacc.at[pl.ds(s * 640, 640)])
    pltpu.sync_copy(ones_hbm, ones_v)
    pltpu.sync_copy(dsti.at[wid], dst_v)
    plsc.subcore_barrier()

    def body(ci, carry):
        pltpu.sync_copy(ones_v, dacc.at[dst_v.at[ci]], add=True)
        return carry

    lax.fori_loop(0, nch, body, 0)
    plsc.subcore_barrier()
    pltpu.sync_copy(dacc.at[pl.ds(s * 640, 640)],
                    out.at[c, 0, pl.ds(s * 640, 640)])


def _sc_degree(cpt0, cpt1, dsti, ones_in, zer_in):
    cptm = dsti.shape[1]
    f = pl.kernel(
        functools.partial(_deg_body, cpt0, cpt1),
        out_type=jax.ShapeDtypeStruct((NC, 1, NPD), F32),
        mesh=_sc_mesh(),
        scratch_types=[
            pltpu.VMEM((cptm, CHUNK), jnp.int32),
            pltpu.VMEM((CHUNK,), F32),
            pltpu.VMEM_SHARED((NPD,), F32),
        ],
    )
    return f(dsti, ones_in, zer_in)


def _agg_body(cpt0, cpt1, rpw, hs, srci, dsti, zer_hbm, out,
              src_v, dst_v, rows0, acc, sem0):
    c = lax.axis_index("c")
    s = lax.axis_index("s")
    wid = s * NC + c
    nch = jnp.where(c == 0, cpt0, cpt1)
    pltpu.sync_copy(zer_hbm, acc.at[pl.ds(s * rpw, rpw)])
    pltpu.sync_copy(srci.at[wid], src_v)
    pltpu.sync_copy(dsti.at[wid], dst_v)
    plsc.subcore_barrier()

    def body(i, carry):
        pltpu.async_copy(hs.at[src_v.at[i]], rows0, sem0).wait()
        pltpu.sync_copy(rows0, acc.at[dst_v.at[i]], add=True)
        return carry

    lax.fori_loop(0, nch, body, 0)
    plsc.subcore_barrier()
    pltpu.sync_copy(acc.at[pl.ds(s * rpw, rpw)],
                    out.at[c, pl.ds(s * rpw, rpw)])


def _sc_aggregate(np_, cpt0, cpt1, hs, srci, dsti, zer_in):
    rpw = np_ // NS
    cptm = srci.shape[1]
    f = pl.kernel(
        functools.partial(_agg_body, cpt0, cpt1, rpw),
        out_type=jax.ShapeDtypeStruct((NC, np_, 128), F32),
        mesh=_sc_mesh(),
        scratch_types=[
            pltpu.VMEM((cptm, CHUNK), jnp.int32),
            pltpu.VMEM((cptm, CHUNK), jnp.int32),
            pltpu.VMEM((CHUNK, 128), F32),
            pltpu.VMEM_SHARED((np_, 128), F32),
            pltpu.SemaphoreType.DMA,
        ],
    )
    return f(hs, srci, dsti, zer_in)


def _gelu(v):
    return 0.5 * v * (1.0 + lax.erf(v * (2.0 ** -0.5)))


def _dot(a, b):
    return jnp.dot(a, b, preferred_element_type=F32,
                   precision=lax.Precision.HIGHEST)


def _scale_body(x_ref, w_ref, d0_ref, d1_ref, hs_ref, dinv_ref):
    deg = d0_ref[0, 0, :] + d1_ref[0, 0, :] + 1.0
    dv = lax.rsqrt(deg)
    dinv_ref[0, 0, :] = dv
    hs_ref[...] = dv[:, None] * _dot(x_ref[...], w_ref[...])


def _tc_scale(np_, nblk, r, xp, w0big, deg0, deg1):
    f = pl.pallas_call(
        _scale_body,
        grid=(nblk,),
        in_specs=[
            pl.BlockSpec((r, 512), lambda i: (i, 0)),
            pl.BlockSpec((512, 128), lambda i: (0, 0)),
            pl.BlockSpec((1, 1, r), lambda i: (i, 0, 0)),
            pl.BlockSpec((1, 1, r), lambda i: (i, 0, 0)),
        ],
        out_specs=[
            pl.BlockSpec((r, 128), lambda i: (i, 0)),
            pl.BlockSpec((1, 1, r), lambda i: (i, 0, 0)),
        ],
        out_shape=[
            jax.ShapeDtypeStruct((np_, 128), F32),
            jax.ShapeDtypeStruct((nblk, 1, r), F32),
        ],
        compiler_params=pltpu.CompilerParams(
            dimension_semantics=("arbitrary",)),
    )
    return f(xp, w0big, deg0, deg1)


def _stats_body(g, r, parts_ref, hs_ref, dinv_ref, bt_ref, batch_ref,
                s_ref, stats_ref):
    i = pl.program_id(0)
    dv = dinv_ref[0, 0, :]
    sv = dv[:, None] * (parts_ref[0] + parts_ref[1] + hs_ref[...]) \
        + bt_ref[0, :][None, :]
    s_ref[...] = sv
    b = batch_ref[0, 0, :]
    oh = (lax.broadcasted_iota(jnp.int32, (g, r), 0) == b[None, :]) \
        .astype(F32)
    a1 = _dot(oh, sv)
    a2 = _dot(oh, sv * sv)
    cnt = jnp.sum(oh, axis=1)
    new = jnp.stack([a1, a2, jnp.broadcast_to(cnt[:, None], (g, 128))])

    @pl.when(i == 0)
    def _():
        stats_ref[...] = new

    @pl.when(i > 0)
    def _():
        stats_ref[...] += new


def _tc_stats(g, np_, nblk, r, parts, hs, dinv, bt, batch3):
    f = pl.pallas_call(
        functools.partial(_stats_body, g, r),
        grid=(nblk,),
        in_specs=[
            pl.BlockSpec((2, r, 128), lambda i: (0, i, 0)),
            pl.BlockSpec((r, 128), lambda i: (i, 0)),
            pl.BlockSpec((1, 1, r), lambda i: (i, 0, 0)),
            pl.BlockSpec((1, 128), lambda i: (0, 0)),
            pl.BlockSpec((1, 1, r), lambda i: (i, 0, 0)),
        ],
        out_specs=[
            pl.BlockSpec((r, 128), lambda i: (i, 0)),
            pl.BlockSpec((3, g, 128), lambda i: (0, 0, 0)),
        ],
        out_shape=[
            jax.ShapeDtypeStruct((np_, 128), F32),
            jax.ShapeDtypeStruct((3, g, 128), F32),
        ],
        compiler_params=pltpu.CompilerParams(
            dimension_semantics=("arbitrary",)),
    )
    return f(parts, hs, dinv, bt, batch3)


def _norm_scale_off(stats, gw, gb, gms):
    cnt = jnp.maximum(stats[2], 1.0)
    mean = stats[0] / cnt
    e2 = stats[1] / cnt
    ms = gms[0, :][None, :]
    var = e2 - mean * mean * ms * (2.0 - ms)
    rs = lax.rsqrt(var + EPS)
    w = gw[0, :][None, :]
    scale = w * rs
    off = gb[0, :][None, :] - w * ms * mean * rs
    return scale, off


def _row_gather(batch_ref, g, r, scale, off):
    b = batch_ref[0, 0, :]
    oht = (b[:, None] == lax.broadcasted_iota(jnp.int32, (r, g), 1)) \
        .astype(F32)
    return _dot(oht, scale), _dot(oht, off)


def _norm0_body(g, r, s_ref, stats_ref, batch_ref, dinv_ref,
                gw_ref, gb_ref, gms_ref, w1_ref, res_ref, hs1_ref):
    scale, off = _norm_scale_off(stats_ref[...], gw_ref, gb_ref, gms_ref)
    sc_r, off_r = _row_gather(batch_ref, g, r, scale, off)
    gv = _gelu(s_ref[...] * sc_r + off_r)
    res_ref[...] = gv
    hs1_ref[...] = dinv_ref[0, 0, :][:, None] * _dot(gv, w1_ref[...])


def _tc_norm0(g, np_, nblk, r, s, stats, batch3, dinv, gw, gb, gms, w1big):
    f = pl.pallas_call(
        functools.partial(_norm0_body, g, r),
        grid=(nblk,),
        in_specs=[
            pl.BlockSpec((r, 128), lambda i: (i, 0)),
            pl.BlockSpec((3, g, 128), lambda i: (0, 0, 0)),
            pl.BlockSpec((1, 1, r), lambda i: (i, 0, 0)),
            pl.BlockSpec((1, 1, r), lambda i: (i, 0, 0)),
            pl.BlockSpec((1, 128), lambda i: (0, 0)),
            pl.BlockSpec((1, 128), lambda i: (0, 0)),
            pl.BlockSpec((1, 128), lambda i: (0, 0)),
            pl.BlockSpec((128, 128), lambda i: (0, 0)),
        ],
        out_specs=[
            pl.BlockSpec((r, 128), lambda i: (i, 0)),
            pl.BlockSpec((r, 128), lambda i: (i, 0)),
        ],
        out_shape=[
            jax.ShapeDtypeStruct((np_, 128), F32),
            jax.ShapeDtypeStruct((np_, 128), F32),
        ],
        compiler_params=pltpu.CompilerParams(
            dimension_semantics=("arbitrary",)),
    )
    return f(s, stats, batch3, dinv, gw, gb, gms, w1big)


def _final_body(g, r, s_ref, stats_ref, batch_ref, res_ref,
                gw_ref, gb_ref, gms_ref, hw0_ref, hb0_ref, hw1_ref, hb1_ref,
                z_ref):
    scale, off = _norm_scale_off(stats_ref[...], gw_ref, gb_ref, gms_ref)
    sc_r, off_r = _row_gather(batch_ref, g, r, scale, off)
    h1 = _gelu(s_ref[...] * sc_r + off_r) + res_ref[...]
    t1 = _gelu(_dot(h1, hw0_ref[...]) + hb0_ref[0, :][None, :])
    z_ref[...] = _dot(t1, hw1_ref[...]) + hb1_ref[0, :][None, :]


def _tc_final(g, np_, nblk, r, s, stats, batch3, res0,
              gw, gb, gms, hw0big, hb0t, hw1big, hb1t):
    f = pl.pallas_call(
        functools.partial(_final_body, g, r),
        grid=(nblk,),
        in_specs=[
            pl.BlockSpec((r, 128), lambda i: (i, 0)),
            pl.BlockSpec((3, g, 128), lambda i: (0, 0, 0)),
            pl.BlockSpec((1, 1, r), lambda i: (i, 0, 0)),
            pl.BlockSpec((r, 128), lambda i: (i, 0)),
            pl.BlockSpec((1, 128), lambda i: (0, 0)),
            pl.BlockSpec((1, 128), lambda i: (0, 0)),
            pl.BlockSpec((1, 128), lambda i: (0, 0)),
            pl.BlockSpec((128, 128), lambda i: (0, 0)),
            pl.BlockSpec((1, 128), lambda i: (0, 0)),
            pl.BlockSpec((128, 128), lambda i: (0, 0)),
            pl.BlockSpec((1, 128), lambda i: (0, 0)),
        ],
        out_specs=[pl.BlockSpec((r, 128), lambda i: (i, 0))],
        out_shape=[jax.ShapeDtypeStruct((np_, 128), F32)],
        compiler_params=pltpu.CompilerParams(
            dimension_semantics=("arbitrary",)),
    )
    return f(s, stats, batch3, res0, gw, gb, gms, hw0big, hb0t, hw1big, hb1t)[0]


def kernel(x, batch, edge_index, W0, b0, gn0_w, gn0_b, gn0_ms,
           W1, b1, gn1_w, gn1_b, gn1_ms, hW0, hb0, hW1, hb1):
    n, t, d = x.shape
    h = W0.shape[1]
    e = edge_index.shape[1]
    g = 16
    out_f = hW1.shape[1]

    r = 1280
    np_ = ((n + 1 + r - 1) // r) * r
    nblk = np_ // r
    ncht = (e + NS * CHUNK - 1) // (NS * CHUNK)
    cpt1 = ncht // 3
    cpt0 = ncht - cpt1
    e_pad = ncht * NS * CHUNK

    x2 = x.reshape(n, t * d)
    xp = jnp.zeros((np_, t * d), F32).at[:n].set(x2)
    batchp = jnp.full((np_,), g, jnp.int32).at[:n].set(batch.astype(jnp.int32))
    batch3 = batchp.reshape(nblk, 1, r)
    ei = edge_index.astype(jnp.int32)
    srcp = jnp.full((e_pad,), n, jnp.int32).at[:e].set(ei[0])
    dstp = jnp.full((e_pad,), n, jnp.int32).at[:e].set(ei[1])

    def _split(flat):
        e0 = NS * cpt0 * CHUNK
        p0 = flat[:e0].reshape(NS, cpt0, CHUNK)
        p1 = flat[e0:].reshape(NS, cpt1, CHUNK)
        p1 = jnp.pad(p1, ((0, 0), (0, cpt0 - cpt1), (0, 0)),
                     constant_values=n)
        return jnp.stack([p0, p1], axis=1).reshape(NW, cpt0, CHUNK)

    srci = _split(srcp)
    dsti = _split(dstp)

    eye_t = jnp.eye(t, dtype=F32)
    w0big = jnp.kron(eye_t, W0)
    w1big = jnp.kron(eye_t, W1)
    hw0big = jnp.kron(eye_t, hW0)
    hw1big = jnp.zeros((t * h, 128), F32).at[:, :t * out_f].set(
        jnp.kron(eye_t, hW1))
    b0t = jnp.tile(b0, t).reshape(1, t * h)
    b1t = jnp.tile(b1, t).reshape(1, t * h)
    gw0 = jnp.tile(gn0_w, t).reshape(1, t * h)
    gb0 = jnp.tile(gn0_b, t).reshape(1, t * h)
    gm0 = jnp.tile(gn0_ms, t).reshape(1, t * h)
    gw1 = jnp.tile(gn1_w, t).reshape(1, t * h)
    gb1 = jnp.tile(gn1_b, t).reshape(1, t * h)
    gm1 = jnp.tile(gn1_ms, t).reshape(1, t * h)
    hb0t = jnp.tile(hb0, t).reshape(1, t * h)
    hb1t = jnp.zeros((1, 128), F32).at[0, :t * out_f].set(jnp.tile(hb1, t))

    rpw = np_ // NS
    zer128 = jnp.zeros((rpw, 128), F32)
    zer1 = jnp.zeros((640,), F32)
    ones1 = jnp.ones((CHUNK,), F32)

    degp = _sc_degree(cpt0, cpt1, dsti, ones1, zer1)
    deg0 = degp[0, 0, :np_].reshape(nblk, 1, r)
    deg1 = degp[1, 0, :np_].reshape(nblk, 1, r)

    hs0, dinv = _tc_scale(np_, nblk, r, xp, w0big, deg0, deg1)
    parts0 = _sc_aggregate(np_, cpt0, cpt1, hs0, srci, dsti, zer128)
    s0, stats0 = _tc_stats(g, np_, nblk, r, parts0, hs0, dinv, b0t, batch3)
    res0, hs1 = _tc_norm0(g, np_, nblk, r, s0, stats0, batch3, dinv,
                          gw0, gb0, gm0, w1big)
    parts1 = _sc_aggregate(np_, cpt0, cpt1, hs1, srci, dsti, zer128)
    s1, stats1 = _tc_stats(g, np_, nblk, r, parts1, hs1, dinv, b1t, batch3)
    z = _tc_final(g, np_, nblk, r, s1, stats1, batch3, res0,
                  gw1, gb1, gm1, hw0big, hb0t, hw1big, hb1t)

    return z[:n, :t * out_f].reshape(n, t, out_f)
